# baseline (speedup 1.0000x reference)
import functools

import jax
import jax.numpy as jnp
from jax.experimental import pallas as pl
from jax.experimental.pallas import tpu as pltpu

_THRESHOLD = 0.15
_BS = 64
_TQ = 256
_NEG = -1e30


def _attn_body(q_ref, k_ref, v_ref, o_ref, c_ref, r_ref, kaug_ref,
               *, n, tq, d, thr):
    nkb = n // _BS
    scale = d ** (-0.5)
    qb = pl.program_id(1)

    @pl.when(qb == 0)
    def _compute_geometry():
        k = k_ref[0]
        sel = (jax.lax.broadcasted_iota(jnp.int32, (nkb, n), 1) // _BS ==
               jax.lax.broadcasted_iota(jnp.int32, (nkb, n), 0))
        selT = (jax.lax.broadcasted_iota(jnp.int32, (n, nkb), 0) // _BS ==
                jax.lax.broadcasted_iota(jnp.int32, (n, nkb), 1))
        kaug_ref[...] = jnp.concatenate(
            [k, jnp.where(selT, (_NEG * -1.0), 0.0),
             jnp.ones((n, 1), jnp.float32)], axis=1)
        c = jax.lax.dot(sel.astype(jnp.float32), k,
                        preferred_element_type=jnp.float32) * (1.0 / _BS)
        c_ref[...] = c
        kc = jax.lax.dot_general(k, c, (((1,), (1,)), ((), ())),
                                 preferred_element_type=jnp.float32)
        k2 = jnp.sum(k * k, axis=1, keepdims=True)
        c2 = jnp.sum(c * c, axis=1)[None, :]
        d2 = k2 - 2.0 * kc + c2
        row_blk = jax.lax.broadcasted_iota(jnp.int32, (n, nkb), 0) // _BS
        col_blk = jax.lax.broadcasted_iota(jnp.int32, (n, nkb), 1)
        d2 = jnp.where(row_blk == col_blk, d2, 0.0)
        r2 = jnp.max(d2, axis=0)[None, :]
        r_ref[...] = jnp.sqrt(jnp.maximum(r2, 0.0))

    q = q_ref[0]
    c = c_ref[...]
    rad = r_ref[...]

    qc = jax.lax.dot_general(q, c, (((1,), (1,)), ((), ())),
                             preferred_element_type=jnp.float32)
    qn = jnp.sqrt(jnp.sum(q * q, axis=1, keepdims=True))
    bound = scale * (qc + qn * rad)
    keep_row = bound >= thr

    row_grp = jax.lax.broadcasted_iota(jnp.int32, (tq, 1), 0) // _BS
    keepmat = jnp.zeros((tq, nkb), jnp.float32)
    for g in range(tq // _BS):
        any_g = jnp.any(keep_row[g * _BS:(g + 1) * _BS, :], axis=0,
                        keepdims=True)
        keepmat = jnp.where(row_grp == g, any_g.astype(jnp.float32), keepmat)
    rowkeep = jnp.max(keepmat, axis=1, keepdims=True) > 0.5

    log2e = 1.4426950408889634
    m = jnp.max(jnp.where(keepmat > 0.5, bound * log2e, _NEG), axis=1,
                keepdims=True)

    v = v_ref[0]
    qaug = jnp.concatenate([q * (scale * log2e), keepmat - 1.0, -m], axis=1)
    s = jax.lax.dot_general(qaug, kaug_ref[...],
                            (((1,), (1,)), ((), ())),
                            preferred_element_type=jnp.float32)
    p = jnp.exp2(s)
    l = jnp.sum(p, axis=1, keepdims=True)
    o = jax.lax.dot(p.astype(jnp.bfloat16), v,
                    preferred_element_type=jnp.float32)
    o_ref[0] = jnp.where(rowkeep, o / l, 0.0)


def _aether(q, k, v, thr):
    b, m, h, d = q.shape
    n = k.shape[1]
    g = b * h
    qg = q.transpose(0, 2, 1, 3).reshape(g, m, d)
    kg = k.transpose(0, 2, 1, 3).reshape(g, n, d)
    vg = v.transpose(0, 2, 1, 3).reshape(g, n, d).astype(jnp.bfloat16)

    nkb = n // _BS
    body = functools.partial(_attn_body, n=n, tq=_TQ, d=d, thr=thr)
    out = pl.pallas_call(
        body,
        grid=(g, m // _TQ),
        in_specs=[
            pl.BlockSpec((1, _TQ, d), lambda i, j: (i, j, 0)),
            pl.BlockSpec((1, n, d), lambda i, j: (i, 0, 0)),
            pl.BlockSpec((1, n, d), lambda i, j: (i, 0, 0)),
        ],
        out_specs=pl.BlockSpec((1, _TQ, d), lambda i, j: (i, j, 0)),
        out_shape=jax.ShapeDtypeStruct((g, m, d), jnp.float32),
        scratch_shapes=[
            pltpu.VMEM((nkb, d), jnp.float32),
            pltpu.VMEM((1, nkb), jnp.float32),
            pltpu.VMEM((n, d + nkb + 1), jnp.float32),
        ],
        compiler_params=pltpu.CompilerParams(
            dimension_semantics=("arbitrary", "arbitrary"),
        ),
    )(qg, kg, vg)
    return out.reshape(b, h, m, d).transpose(0, 2, 1, 3)


def kernel(q, k, v):
    return _aether(q, k, v, _THRESHOLD)


# speedup vs baseline: 1.0022x; 1.0022x over previous
import functools

import jax
import jax.numpy as jnp
from jax.experimental import pallas as pl
from jax.experimental.pallas import tpu as pltpu

_THRESHOLD = 0.15
_BS = 64
_TQ = 256
_NEG = -1e30


def _attn_body(q_ref, k_ref, v_ref, o_ref, c_ref, r_ref, kaug_ref,
               *, n, tq, d, thr):
    nkb = n // _BS
    scale = d ** (-0.5)
    qb = pl.program_id(1)

    @pl.when(qb == 0)
    def _compute_geometry():
        k = k_ref[0]
        sel = (jax.lax.broadcasted_iota(jnp.int32, (nkb, n), 1) // _BS ==
               jax.lax.broadcasted_iota(jnp.int32, (nkb, n), 0))
        selT = (jax.lax.broadcasted_iota(jnp.int32, (n, nkb), 0) // _BS ==
                jax.lax.broadcasted_iota(jnp.int32, (n, nkb), 1))
        kaug_ref[...] = jnp.concatenate(
            [k, jnp.where(selT, (_NEG * -1.0), 0.0),
             jnp.ones((n, 1), jnp.float32),
             jnp.zeros((n, 31), jnp.float32)], axis=1)
        c = jax.lax.dot(sel.astype(jnp.float32), k,
                        preferred_element_type=jnp.float32) * (1.0 / _BS)
        c_ref[...] = c
        kc = jax.lax.dot_general(k, c, (((1,), (1,)), ((), ())),
                                 preferred_element_type=jnp.float32)
        k2 = jnp.sum(k * k, axis=1, keepdims=True)
        c2 = jnp.sum(c * c, axis=1)[None, :]
        d2 = k2 - 2.0 * kc + c2
        row_blk = jax.lax.broadcasted_iota(jnp.int32, (n, nkb), 0) // _BS
        col_blk = jax.lax.broadcasted_iota(jnp.int32, (n, nkb), 1)
        d2 = jnp.where(row_blk == col_blk, d2, 0.0)
        r2 = jnp.max(d2, axis=0)[None, :]
        r_ref[...] = jnp.sqrt(jnp.maximum(r2, 0.0))

    q = q_ref[0]
    c = c_ref[...]
    rad = r_ref[...]

    qc = jax.lax.dot_general(q, c, (((1,), (1,)), ((), ())),
                             preferred_element_type=jnp.float32)
    qn = jnp.sqrt(jnp.sum(q * q, axis=1, keepdims=True))
    bound = scale * (qc + qn * rad)
    keep_row = bound >= thr

    row_grp = jax.lax.broadcasted_iota(jnp.int32, (tq, 1), 0) // _BS
    keepmat = jnp.zeros((tq, nkb), jnp.float32)
    for g in range(tq // _BS):
        any_g = jnp.any(keep_row[g * _BS:(g + 1) * _BS, :], axis=0,
                        keepdims=True)
        keepmat = jnp.where(row_grp == g, any_g.astype(jnp.float32), keepmat)
    rowkeep = jnp.max(keepmat, axis=1, keepdims=True) > 0.5

    log2e = 1.4426950408889634
    m = jnp.max(jnp.where(keepmat > 0.5, bound * log2e, _NEG), axis=1,
                keepdims=True)

    v = v_ref[0]
    qaug = jnp.concatenate([q * (scale * log2e), keepmat - 1.0, -m,
                            jnp.zeros((tq, 31), jnp.float32)], axis=1)
    s = jax.lax.dot_general(qaug, kaug_ref[...],
                            (((1,), (1,)), ((), ())),
                            preferred_element_type=jnp.float32)
    p = jnp.exp2(s)
    l = jnp.sum(p, axis=1, keepdims=True)
    o = jax.lax.dot(p.astype(jnp.bfloat16), v,
                    preferred_element_type=jnp.float32)
    o_ref[0] = jnp.where(rowkeep, o / l, 0.0)


def _aether(q, k, v, thr):
    b, m, h, d = q.shape
    n = k.shape[1]
    g = b * h
    qg = q.transpose(0, 2, 1, 3).reshape(g, m, d)
    kg = k.transpose(0, 2, 1, 3).reshape(g, n, d)
    vg = v.transpose(0, 2, 1, 3).reshape(g, n, d).astype(jnp.bfloat16)

    nkb = n // _BS
    body = functools.partial(_attn_body, n=n, tq=_TQ, d=d, thr=thr)
    out = pl.pallas_call(
        body,
        grid=(g, m // _TQ),
        in_specs=[
            pl.BlockSpec((1, _TQ, d), lambda i, j: (i, j, 0)),
            pl.BlockSpec((1, n, d), lambda i, j: (i, 0, 0)),
            pl.BlockSpec((1, n, d), lambda i, j: (i, 0, 0)),
        ],
        out_specs=pl.BlockSpec((1, _TQ, d), lambda i, j: (i, j, 0)),
        out_shape=jax.ShapeDtypeStruct((g, m, d), jnp.float32),
        scratch_shapes=[
            pltpu.VMEM((nkb, d), jnp.float32),
            pltpu.VMEM((1, nkb), jnp.float32),
            pltpu.VMEM((n, d + nkb + 32), jnp.float32),
        ],
        compiler_params=pltpu.CompilerParams(
            dimension_semantics=("arbitrary", "arbitrary"),
        ),
    )(qg, kg, vg)
    return out.reshape(b, h, m, d).transpose(0, 2, 1, 3)


def kernel(q, k, v):
    return _aether(q, k, v, _THRESHOLD)


# speedup vs baseline: 1.1497x; 1.1473x over previous
import functools

import jax
import jax.numpy as jnp
from jax.experimental import pallas as pl
from jax.experimental.pallas import tpu as pltpu

_THRESHOLD = 0.15
_BS = 64
_TQ = 256
_NEG = -1e30


def _attn_body(q_ref, k_ref, v_ref, o_ref, c_ref, r_ref, kaug_ref,
               *, n, tq, d, thr):
    nkb = n // _BS
    scale = d ** (-0.5)
    qb = pl.program_id(1)

    @pl.when(qb == 0)
    def _compute_geometry():
        k = k_ref[0]
        sel = (jax.lax.broadcasted_iota(jnp.int32, (nkb, n), 1) // _BS ==
               jax.lax.broadcasted_iota(jnp.int32, (nkb, n), 0))
        selT = (jax.lax.broadcasted_iota(jnp.int32, (n, nkb), 0) // _BS ==
                jax.lax.broadcasted_iota(jnp.int32, (n, nkb), 1))
        kaug_ref[...] = jnp.concatenate(
            [k, jnp.where(selT, (_NEG * -1.0), 0.0)], axis=1)
        c = jax.lax.dot(sel.astype(jnp.float32), k,
                        preferred_element_type=jnp.float32) * (1.0 / _BS)
        c_ref[...] = c
        kc = jax.lax.dot_general(k, c, (((1,), (1,)), ((), ())),
                                 preferred_element_type=jnp.float32)
        k2 = jnp.sum(k * k, axis=1, keepdims=True)
        c2 = jnp.sum(c * c, axis=1)[None, :]
        d2 = k2 - 2.0 * kc + c2
        row_blk = jax.lax.broadcasted_iota(jnp.int32, (n, nkb), 0) // _BS
        col_blk = jax.lax.broadcasted_iota(jnp.int32, (n, nkb), 1)
        d2 = jnp.where(row_blk == col_blk, d2, 0.0)
        r2 = jnp.max(d2, axis=0)[None, :]
        r_ref[...] = jnp.sqrt(jnp.maximum(r2, 0.0))

    q = q_ref[0]
    c = c_ref[...]
    rad = r_ref[...]

    qc = jax.lax.dot_general(q, c, (((1,), (1,)), ((), ())),
                             preferred_element_type=jnp.float32)
    qn = jnp.sqrt(jnp.sum(q * q, axis=1, keepdims=True))
    bound = scale * (qc + qn * rad)
    keep_row = bound >= thr

    row_grp = jax.lax.broadcasted_iota(jnp.int32, (tq, 1), 0) // _BS
    keepmat = jnp.zeros((tq, nkb), jnp.float32)
    for g in range(tq // _BS):
        any_g = jnp.any(keep_row[g * _BS:(g + 1) * _BS, :], axis=0,
                        keepdims=True)
        keepmat = jnp.where(row_grp == g, any_g.astype(jnp.float32), keepmat)
    rowkeep = jnp.max(keepmat, axis=1, keepdims=True) > 0.5

    log2e = 1.4426950408889634
    m = jnp.max(jnp.where(keepmat > 0.5, bound * log2e, _NEG), axis=1,
                keepdims=True)

    v = v_ref[0]
    qaug = jnp.concatenate([q * (scale * log2e), keepmat - 1.0], axis=1)
    s = jax.lax.dot_general(qaug, kaug_ref[...],
                            (((1,), (1,)), ((), ())),
                            preferred_element_type=jnp.float32)
    p = jnp.exp2(s - m)
    l = jnp.sum(p, axis=1, keepdims=True)
    o = jax.lax.dot(p.astype(jnp.bfloat16), v,
                    preferred_element_type=jnp.float32)
    o_ref[0] = jnp.where(rowkeep, o / l, 0.0)


def _aether(q, k, v, thr):
    b, m, h, d = q.shape
    n = k.shape[1]
    g = b * h
    qg = q.transpose(0, 2, 1, 3).reshape(g, m, d)
    kg = k.transpose(0, 2, 1, 3).reshape(g, n, d)
    vg = v.transpose(0, 2, 1, 3).reshape(g, n, d).astype(jnp.bfloat16)

    nkb = n // _BS
    body = functools.partial(_attn_body, n=n, tq=_TQ, d=d, thr=thr)
    out = pl.pallas_call(
        body,
        grid=(g, m // _TQ),
        in_specs=[
            pl.BlockSpec((1, _TQ, d), lambda i, j: (i, j, 0)),
            pl.BlockSpec((1, n, d), lambda i, j: (i, 0, 0)),
            pl.BlockSpec((1, n, d), lambda i, j: (i, 0, 0)),
        ],
        out_specs=pl.BlockSpec((1, _TQ, d), lambda i, j: (i, j, 0)),
        out_shape=jax.ShapeDtypeStruct((g, m, d), jnp.float32),
        scratch_shapes=[
            pltpu.VMEM((nkb, d), jnp.float32),
            pltpu.VMEM((1, nkb), jnp.float32),
            pltpu.VMEM((n, d + nkb), jnp.float32),
        ],
        compiler_params=pltpu.CompilerParams(
            dimension_semantics=("arbitrary", "arbitrary"),
        ),
    )(qg, kg, vg)
    return out.reshape(b, h, m, d).transpose(0, 2, 1, 3)


def kernel(q, k, v):
    return _aether(q, k, v, _THRESHOLD)


# speedup vs baseline: 1.2443x; 1.0823x over previous
import functools

import jax
import jax.numpy as jnp
from jax.experimental import pallas as pl
from jax.experimental.pallas import tpu as pltpu

_THRESHOLD = 0.15
_BS = 64
_TQ = 256
_NEG = -1e30
_LOG2E = 1.4426950408889634


def _attn_body(q_ref, k_ref, v_ref, o_ref, c_ref, r_ref, kaug_ref, v16_ref,
               *, n, tq, d, thr):
    nkb = n // _BS
    scale = d ** (-0.5)
    qb = pl.program_id(1)

    @pl.when(qb == 0)
    def _compute_geometry():
        sel = (jax.lax.broadcasted_iota(jnp.int32, (nkb, n), 1) // _BS ==
               jax.lax.broadcasted_iota(jnp.int32, (nkb, n), 0))
        selT = (jax.lax.broadcasted_iota(jnp.int32, (n, nkb), 0) // _BS ==
                jax.lax.broadcasted_iota(jnp.int32, (n, nkb), 1))
        onehot_cols = jnp.where(selT, (_NEG * -1.0), 0.0)
        row_blk = jax.lax.broadcasted_iota(jnp.int32, (n, nkb), 0) // _BS
        col_blk = jax.lax.broadcasted_iota(jnp.int32, (n, nkb), 1)
        for hh in range(2):
            k = k_ref[0, :, hh * d:(hh + 1) * d]
            kaug_ref[hh] = jnp.concatenate([k, onehot_cols], axis=1)
            v16_ref[hh] = v_ref[0, :, hh * d:(hh + 1) * d].astype(
                jnp.bfloat16)
            c = jax.lax.dot(sel.astype(jnp.float32), k,
                            preferred_element_type=jnp.float32) * (1.0 / _BS)
            c_ref[hh] = c
            kc = jax.lax.dot_general(k, c, (((1,), (1,)), ((), ())),
                                     preferred_element_type=jnp.float32)
            k2 = jnp.sum(k * k, axis=1, keepdims=True)
            c2 = jnp.sum(c * c, axis=1)[None, :]
            d2 = k2 - 2.0 * kc + c2
            d2 = jnp.where(row_blk == col_blk, d2, 0.0)
            r2 = jnp.max(d2, axis=0)[None, :]
            r_ref[hh] = jnp.sqrt(jnp.maximum(r2, 0.0))

    for hh in range(2):
        q = q_ref[0, :, hh * d:(hh + 1) * d]
        c = c_ref[hh]
        rad = r_ref[hh]

        qc = jax.lax.dot_general(q, c, (((1,), (1,)), ((), ())),
                                 preferred_element_type=jnp.float32)
        qn = jnp.sqrt(jnp.sum(q * q, axis=1, keepdims=True))
        bound = scale * (qc + qn * rad)
        keep_row = bound >= thr

        row_grp = jax.lax.broadcasted_iota(jnp.int32, (tq, 1), 0) // _BS
        keepmat = jnp.zeros((tq, nkb), jnp.float32)
        for g in range(tq // _BS):
            any_g = jnp.any(keep_row[g * _BS:(g + 1) * _BS, :], axis=0,
                            keepdims=True)
            keepmat = jnp.where(row_grp == g, any_g.astype(jnp.float32),
                                keepmat)
        rowkeep = jnp.max(keepmat, axis=1, keepdims=True) > 0.5

        m = jnp.max(jnp.where(keepmat > 0.5, bound * _LOG2E, _NEG), axis=1,
                    keepdims=True)

        qaug = jnp.concatenate([q * (scale * _LOG2E), keepmat - 1.0], axis=1)
        s = jax.lax.dot_general(qaug, kaug_ref[hh],
                                (((1,), (1,)), ((), ())),
                                preferred_element_type=jnp.float32)
        p = jnp.exp2(s - m)
        l = jnp.sum(p, axis=1, keepdims=True)
        o = jax.lax.dot(p.astype(jnp.bfloat16), v16_ref[hh],
                        preferred_element_type=jnp.float32)
        o_ref[0, :, hh * d:(hh + 1) * d] = jnp.where(rowkeep, o / l, 0.0)


def _aether(q, k, v, thr):
    b, m, h, d = q.shape
    n = k.shape[1]
    hp = h // 2
    q3 = q.reshape(b, m, h * d)
    k3 = k.reshape(b, n, h * d)
    v3 = v.reshape(b, n, h * d)

    nkb = n // _BS
    body = functools.partial(_attn_body, n=n, tq=_TQ, d=d, thr=thr)
    out = pl.pallas_call(
        body,
        grid=(b * hp, m // _TQ),
        in_specs=[
            pl.BlockSpec((1, _TQ, 2 * d), lambda i, j: (i // hp, j, i % hp)),
            pl.BlockSpec((1, n, 2 * d), lambda i, j: (i // hp, 0, i % hp)),
            pl.BlockSpec((1, n, 2 * d), lambda i, j: (i // hp, 0, i % hp)),
        ],
        out_specs=pl.BlockSpec((1, _TQ, 2 * d),
                               lambda i, j: (i // hp, j, i % hp)),
        out_shape=jax.ShapeDtypeStruct((b, m, h * d), jnp.float32),
        scratch_shapes=[
            pltpu.VMEM((2, nkb, d), jnp.float32),
            pltpu.VMEM((2, 1, nkb), jnp.float32),
            pltpu.VMEM((2, n, d + nkb), jnp.float32),
            pltpu.VMEM((2, n, d), jnp.bfloat16),
        ],
        compiler_params=pltpu.CompilerParams(
            dimension_semantics=("arbitrary", "arbitrary"),
        ),
    )(q3, k3, v3)
    return out.reshape(b, m, h, d)


def kernel(q, k, v):
    return _aether(q, k, v, _THRESHOLD)


# speedup vs baseline: 1.2935x; 1.0395x over previous
import functools

import jax
import jax.numpy as jnp
from jax.experimental import pallas as pl
from jax.experimental.pallas import tpu as pltpu

_THRESHOLD = 0.15
_BS = 64
_TQ = 256
_NEG = -1e30
_LOG2E = 1.4426950408889634
_HPS = 4


def _attn_body(q_ref, k_ref, v_ref, o_ref, c_ref, r_ref, kaug_ref, v16_ref,
               *, n, tq, d, thr):
    nkb = n // _BS
    scale = d ** (-0.5)
    qb = pl.program_id(1)

    @pl.when(qb == 0)
    def _compute_geometry():
        sel = (jax.lax.broadcasted_iota(jnp.int32, (nkb, n), 1) // _BS ==
               jax.lax.broadcasted_iota(jnp.int32, (nkb, n), 0))
        selT = (jax.lax.broadcasted_iota(jnp.int32, (n, nkb), 0) // _BS ==
                jax.lax.broadcasted_iota(jnp.int32, (n, nkb), 1))
        onehot_cols = jnp.where(selT, (_NEG * -1.0), 0.0)
        row_blk = jax.lax.broadcasted_iota(jnp.int32, (n, nkb), 0) // _BS
        col_blk = jax.lax.broadcasted_iota(jnp.int32, (n, nkb), 1)
        for hh in range(_HPS):
            k = k_ref[0, :, hh * d:(hh + 1) * d]
            kaug_ref[hh] = jnp.concatenate([k, onehot_cols], axis=1)
            v16_ref[hh] = v_ref[0, :, hh * d:(hh + 1) * d].astype(
                jnp.bfloat16)
            c = jax.lax.dot(sel.astype(jnp.float32), k,
                            preferred_element_type=jnp.float32) * (1.0 / _BS)
            c_ref[hh] = c
            kc = jax.lax.dot_general(k, c, (((1,), (1,)), ((), ())),
                                     preferred_element_type=jnp.float32)
            k2 = jnp.sum(k * k, axis=1, keepdims=True)
            c2 = jnp.sum(c * c, axis=1)[None, :]
            d2 = k2 - 2.0 * kc + c2
            d2 = jnp.where(row_blk == col_blk, d2, 0.0)
            r2 = jnp.max(d2, axis=0)[None, :]
            r_ref[hh] = jnp.sqrt(jnp.maximum(r2, 0.0))

    for hh in range(_HPS):
        q = q_ref[0, :, hh * d:(hh + 1) * d]
        c = c_ref[hh]
        rad = r_ref[hh]

        qc = jax.lax.dot_general(q, c, (((1,), (1,)), ((), ())),
                                 preferred_element_type=jnp.float32)
        qn = jnp.sqrt(jnp.sum(q * q, axis=1, keepdims=True))
        bound = scale * (qc + qn * rad)
        keep_row = bound >= thr

        row_grp = jax.lax.broadcasted_iota(jnp.int32, (tq, 1), 0) // _BS
        keepmat = jnp.zeros((tq, nkb), jnp.float32)
        for g in range(tq // _BS):
            any_g = jnp.any(keep_row[g * _BS:(g + 1) * _BS, :], axis=0,
                            keepdims=True)
            keepmat = jnp.where(row_grp == g, any_g.astype(jnp.float32),
                                keepmat)
        rowkeep = jnp.max(keepmat, axis=1, keepdims=True) > 0.5

        m = jnp.max(jnp.where(keepmat > 0.5, bound * _LOG2E, _NEG), axis=1,
                    keepdims=True)

        qaug = jnp.concatenate([q * (scale * _LOG2E), keepmat - 1.0], axis=1)
        s = jax.lax.dot_general(qaug, kaug_ref[hh],
                                (((1,), (1,)), ((), ())),
                                preferred_element_type=jnp.float32)
        p = jnp.exp2(s - m)
        l = jnp.sum(p, axis=1, keepdims=True)
        o = jax.lax.dot(p.astype(jnp.bfloat16), v16_ref[hh],
                        preferred_element_type=jnp.float32)
        o_ref[0, :, hh * d:(hh + 1) * d] = jnp.where(rowkeep, o / l, 0.0)


def _aether(q, k, v, thr):
    b, m, h, d = q.shape
    n = k.shape[1]
    hp = h // _HPS
    q3 = q.reshape(b, m, h * d)
    k3 = k.reshape(b, n, h * d)
    v3 = v.reshape(b, n, h * d)

    nkb = n // _BS
    body = functools.partial(_attn_body, n=n, tq=_TQ, d=d, thr=thr)
    out = pl.pallas_call(
        body,
        grid=(b * hp, m // _TQ),
        in_specs=[
            pl.BlockSpec((1, _TQ, _HPS * d), lambda i, j: (i // hp, j, i % hp)),
            pl.BlockSpec((1, n, _HPS * d), lambda i, j: (i // hp, 0, i % hp)),
            pl.BlockSpec((1, n, _HPS * d), lambda i, j: (i // hp, 0, i % hp)),
        ],
        out_specs=pl.BlockSpec((1, _TQ, _HPS * d),
                               lambda i, j: (i // hp, j, i % hp)),
        out_shape=jax.ShapeDtypeStruct((b, m, h * d), jnp.float32),
        scratch_shapes=[
            pltpu.VMEM((_HPS, nkb, d), jnp.float32),
            pltpu.VMEM((_HPS, 1, nkb), jnp.float32),
            pltpu.VMEM((_HPS, n, d + nkb), jnp.float32),
            pltpu.VMEM((_HPS, n, d), jnp.bfloat16),
        ],
        compiler_params=pltpu.CompilerParams(
            dimension_semantics=("arbitrary", "arbitrary"),
        ),
    )(q3, k3, v3)
    return out.reshape(b, m, h, d)


def kernel(q, k, v):
    return _aether(q, k, v, _THRESHOLD)
